# Initial kernel scaffold; baseline (speedup 1.0000x reference)
import jax
import jax.numpy as jnp
from jax.experimental import pallas as pl


def kernel(important_token_states, importance_mask, coarse_token_states, coarse_token_mask, important_token_positions, coarse_token_positions, Wq, Wk):
    raise NotImplementedError("write your pallas kernel here")


# speedup vs baseline: 2.8391x; 2.8391x over previous
import jax
import jax.numpy as jnp
from jax.experimental import pallas as pl
from jax.experimental.pallas import tpu as pltpu

_N_HEADS = 12
_NUM_FINE = 64


def _selector_body(imp_last_ref, coarse_ref, wq_ref, wk_ref, bi_ref, sc_ref,
                   ps_ref):
    B, D = imp_last_ref.shape
    N = coarse_ref.shape[1]
    dh = D // _N_HEADS
    q = jnp.dot(imp_last_ref[...], wq_ref[...],
                preferred_element_type=jnp.float32)

    i_iota = jax.lax.broadcasted_iota(jnp.int32, (N, N), 0)
    j_iota = jax.lax.broadcasted_iota(jnp.int32, (N, N), 1)
    j_row = jax.lax.broadcasted_iota(jnp.int32, (1, N), 1)
    i_row_f = j_row.astype(jnp.float32)

    for b in range(B):
        k = jnp.dot(coarse_ref[b], wk_ref[...],
                    preferred_element_type=jnp.float32)
        rows = []
        for h in range(_N_HEADS):
            qh = q[b:b + 1, h * dh:(h + 1) * dh]
            kh = k[:, h * dh:(h + 1) * dh]
            rows.append(jnp.dot(qh, kh.T, preferred_element_type=jnp.float32))
        s = jnp.concatenate(rows, axis=0) / jnp.sqrt(jnp.float32(dh))
        probs = jax.nn.softmax(s, axis=-1)
        logits = jnp.mean(probs, axis=0, keepdims=True)
        p = jax.nn.softmax(logits, axis=-1)

        lrow = jnp.broadcast_to(logits, (N, N))
        lcol = lrow.T
        cmp = (lrow > lcol) | ((lrow == lcol) & (j_iota < i_iota))
        rank = jnp.sum(cmp.astype(jnp.int32), axis=1, keepdims=True)

        m = (jnp.broadcast_to(rank, (N, N)) == j_iota).astype(jnp.float32)
        bi = jnp.dot(i_row_f, m, preferred_element_type=jnp.float32)
        ps = jnp.dot(p, m, preferred_element_type=jnp.float32)

        fine_sc = (1.0 + ps) - ps
        cs = 1.0 - ps
        coarse_sc = (1.0 + cs) - cs
        sc = jnp.where(j_row < _NUM_FINE, fine_sc, coarse_sc)

        bi_ref[b:b + 1, :] = bi.astype(jnp.int32)
        sc_ref[b:b + 1, :] = sc
        ps_ref[b:b + 1, :] = ps


def kernel(important_token_states, importance_mask, coarse_token_states,
           coarse_token_mask, important_token_positions,
           coarse_token_positions, Wq, Wk):
    del importance_mask, coarse_token_mask
    del important_token_positions, coarse_token_positions
    B, _, D = important_token_states.shape
    N = coarse_token_states.shape[1]
    imp_last = important_token_states[:, -1, :]

    bi, sc, _ps = pl.pallas_call(
        _selector_body,
        out_shape=(
            jax.ShapeDtypeStruct((B, N), jnp.int32),
            jax.ShapeDtypeStruct((B, N), jnp.float32),
            jax.ShapeDtypeStruct((B, N), jnp.float32),
        ),
    )(imp_last, coarse_token_states, Wq, Wk)

    fine_block_indices = bi[:, :_NUM_FINE]
    coarse_block_indices = bi[:, _NUM_FINE:]
    fine_block_scores = sc[:, :_NUM_FINE]
    coarse_block_scores = sc[:, _NUM_FINE:]
    return (fine_block_indices, coarse_block_indices, fine_block_scores,
            coarse_block_scores)
